# baseline (speedup 1.0000x reference)
import jax
import jax.numpy as jnp
from jax.experimental import pallas as pl

B, C_IN, HW = 4, 96, 384
ENC, P = 32, 1024
CROP = 3
ROWS = 24
NT = HW // ROWS


def _conv_kernel(fm_ref, w_ref, b_ref, pf_ref):
    fm = fm_ref[0].reshape(C_IN, ROWS * HW).astype(jnp.bfloat16)
    pf = jax.lax.dot_general(
        fm, w_ref[...].astype(jnp.bfloat16), (((0,), (0,)), ((), ())),
        preferred_element_type=jnp.float32)
    pf_ref[0] = pf + b_ref[0][None, :]


def _conv(imgBatch, W, b):
    return pl.pallas_call(
        _conv_kernel,
        grid=(B, NT),
        in_specs=[
            pl.BlockSpec((1, C_IN, ROWS, HW), lambda bi, ti: (bi, 0, ti, 0)),
            pl.BlockSpec((C_IN, ENC), lambda bi, ti: (0, 0)),
            pl.BlockSpec((1, ENC), lambda bi, ti: (0, 0)),
        ],
        out_specs=pl.BlockSpec((1, ROWS * HW, ENC), lambda bi, ti: (bi, ti, 0)),
        out_shape=jax.ShapeDtypeStruct((B, HW * HW, ENC), jnp.float32),
    )(imgBatch, W, b.reshape(1, ENC))


def kernel(imgBatch, W, b):
    pfm = _conv(imgBatch, W, b)
    x = jnp.sum(pfm * pfm, axis=-1)
    hh = jnp.arange(HW * HW, dtype=jnp.int32) // HW
    ww = jnp.arange(HW * HW, dtype=jnp.int32) % HW
    valid = ((hh >= CROP) & (hh < HW - CROP) &
             (ww >= CROP) & (ww < HW - CROP))
    x = jnp.where(valid[None, :], x, -1.0)
    _, pidx = jax.lax.top_k(x, P)
    ordp = pidx // HW - CROP
    absp = pidx % HW - CROP
    feats = jnp.take_along_axis(pfm, pidx[..., None], axis=1)
    absf = absp[..., None].astype(jnp.float32)
    ordf = ordp[..., None].astype(jnp.float32)
    depth = jnp.zeros((B, P, 1), jnp.float32)
    points_full = jnp.concatenate([absf, ordf, depth, feats], axis=-1)
    batch = jnp.broadcast_to(jnp.arange(B)[:, None], (B, P)).reshape(-1)
    pos = jnp.concatenate([absf, ordf, depth], axis=-1).reshape(B * P, 3)
    pointfeatures = feats.reshape(B * P, ENC)
    return (points_full, batch, pos, pointfeatures)


# speedup vs baseline: 1.0033x; 1.0033x over previous
import jax
import jax.numpy as jnp
from jax.experimental import pallas as pl

B, C_IN, HW = 4, 96, 384
ENC, P = 32, 1024
CROP = 3
ROWS = 48
NT = HW // ROWS


def _conv_kernel(fm_ref, w_ref, b_ref, pf_ref):
    fm = fm_ref[0].reshape(C_IN, ROWS * HW).astype(jnp.bfloat16)
    pf = jax.lax.dot_general(
        fm, w_ref[...].astype(jnp.bfloat16), (((0,), (0,)), ((), ())),
        preferred_element_type=jnp.float32)
    pf_ref[0] = pf + b_ref[0][None, :]


def _conv(imgBatch, W, b):
    return pl.pallas_call(
        _conv_kernel,
        grid=(B, NT),
        in_specs=[
            pl.BlockSpec((1, C_IN, ROWS, HW), lambda bi, ti: (bi, 0, ti, 0)),
            pl.BlockSpec((C_IN, ENC), lambda bi, ti: (0, 0)),
            pl.BlockSpec((1, ENC), lambda bi, ti: (0, 0)),
        ],
        out_specs=pl.BlockSpec((1, ROWS * HW, ENC), lambda bi, ti: (bi, ti, 0)),
        out_shape=jax.ShapeDtypeStruct((B, HW * HW, ENC), jnp.float32),
    )(imgBatch, W, b.reshape(1, ENC))


def kernel(imgBatch, W, b):
    pfm = _conv(imgBatch, W, b)
    x = jnp.sum(pfm * pfm, axis=-1)
    hh = jnp.arange(HW * HW, dtype=jnp.int32) // HW
    ww = jnp.arange(HW * HW, dtype=jnp.int32) % HW
    valid = ((hh >= CROP) & (hh < HW - CROP) &
             (ww >= CROP) & (ww < HW - CROP))
    x = jnp.where(valid[None, :], x, -1.0)
    _, pidx = jax.lax.top_k(x, P)
    ordp = pidx // HW - CROP
    absp = pidx % HW - CROP
    feats = jnp.take_along_axis(pfm, pidx[..., None], axis=1)
    absf = absp[..., None].astype(jnp.float32)
    ordf = ordp[..., None].astype(jnp.float32)
    depth = jnp.zeros((B, P, 1), jnp.float32)
    points_full = jnp.concatenate([absf, ordf, depth, feats], axis=-1)
    batch = jnp.broadcast_to(jnp.arange(B)[:, None], (B, P)).reshape(-1)
    pos = jnp.concatenate([absf, ordf, depth], axis=-1).reshape(B * P, 3)
    pointfeatures = feats.reshape(B * P, ENC)
    return (points_full, batch, pos, pointfeatures)
